# baseline (speedup 1.0000x reference)
import functools

import jax
import jax.numpy as jnp
from jax.experimental import pallas as pl
from jax.experimental.pallas import tpu as pltpu


def _se_pipe_step(x_ref, w1t_ref, w2t_ref, o_ref, slab_ref, acc_ref, y_ref,
                  *, inv_hw, bp, nt, s):
    b = pl.program_id(1)
    t = pl.program_id(2)
    par = jax.lax.rem(b, 2)
    prv = jax.lax.rem(b + 1, 2)

    @pl.when(b < bp)
    def _pool():
        blk = x_ref[0]
        psum = jnp.sum(blk, axis=0, keepdims=True)

        @pl.when(t == 0)
        def _init():
            acc_ref[...] = psum

        @pl.when(t != 0)
        def _accum():
            acc_ref[...] = acc_ref[...] + psum

        slab_ref[par, pl.ds(t * s, s), :] = blk

        @pl.when(t == nt - 1)
        def _fc():
            avg = acc_ref[...] * inv_hw
            h = jnp.maximum(
                jnp.dot(avg, w1t_ref[...],
                        preferred_element_type=jnp.float32), 0.0)
            y_ref[par] = jax.nn.sigmoid(
                jnp.dot(h, w2t_ref[...],
                        preferred_element_type=jnp.float32))

    @pl.when(b > 0)
    def _rescale():
        o_ref[0] = slab_ref[prv, pl.ds(t * s, s), :] * y_ref[prv]


def kernel(x_nchw, w1, w2):
    B, C, H, W = x_nchw.shape
    HW = H * W
    Cr = w1.shape[0]
    ncores = 2
    bp = B // ncores
    nt = 4
    s = HW // nt

    x_flat = jnp.transpose(x_nchw, (0, 2, 3, 1)).reshape(B, HW, C)
    w1t = w1.T
    w2t = w2.T

    out_flat = pl.pallas_call(
        functools.partial(_se_pipe_step, inv_hw=1.0 / float(HW),
                          bp=bp, nt=nt, s=s),
        out_shape=jax.ShapeDtypeStruct((B, HW, C), x_nchw.dtype),
        grid=(ncores, bp + 1, nt),
        in_specs=[
            pl.BlockSpec((1, s, C),
                         lambda k, b, t: (k * (B // 2) + jnp.minimum(b, B // 2 - 1), t, 0)),
            pl.BlockSpec((C, Cr), lambda k, b, t: (0, 0)),
            pl.BlockSpec((Cr, C), lambda k, b, t: (0, 0)),
        ],
        out_specs=pl.BlockSpec(
            (1, s, C),
            lambda k, b, t: (k * (B // 2) + jnp.maximum(b - 1, 0), t, 0)),
        scratch_shapes=[
            pltpu.VMEM((2, HW, C), jnp.float32),
            pltpu.VMEM((1, C), jnp.float32),
            pltpu.VMEM((2, 1, C), jnp.float32),
        ],
        compiler_params=pltpu.CompilerParams(
            dimension_semantics=("parallel", "arbitrary", "arbitrary"),
            vmem_limit_bytes=64 << 20),
    )(x_flat, w1t, w2t)

    return jnp.transpose(out_flat.reshape(B, H, W, C), (0, 3, 1, 2))
